# baseline (speedup 1.0000x reference)
import jax
import jax.numpy as jnp
from jax import lax
from jax.experimental import pallas as pl
from jax.experimental.pallas import tpu as pltpu
from jax.experimental.pallas import tpu_sc as plsc

_LOSS_W = 0.5
_ORTH_MU = 0.1
_NC = 2
_LANES = 16

_L = 12
_B = 64
_D = 768
_TOPK = 5
_NCH = _D // _LANES
_BH = _B // 2


def _splat(s):
    return lax.broadcast_in_dim(s, (_LANES,), ())


def _rsqrt_vec(v):
    i = plsc.bitcast(v, jnp.int32)
    i = jnp.int32(0x5F3759DF) - lax.shift_right_logical(
        i, jnp.full((_LANES,), 1, jnp.int32))
    y = plsc.bitcast(i, jnp.float32)
    for _ in range(3):
        y = y * (jnp.float32(1.5) - jnp.float32(0.5) * v * y * y)
    return y


def _lane_sum(v):
    return _splat(jnp.sum(v))


def _sc_loss_body(x_hbm, k_hbm, a_hbm, out_hbm,
                  xbuf, kbuf, abuf, akbuf, a2buf, numbuf, denbuf,
                  partbuf, gathbuf, outbuf, shared):
    c = lax.axis_index("c")
    s = lax.axis_index("s")
    zero = jnp.zeros((_LANES,), jnp.float32)

    @pl.when(s < _L)
    def _compute():
        b0 = c * _BH
        pltpu.sync_copy(x_hbm.at[pl.ds(b0, _BH), s], xbuf)
        pltpu.sync_copy(k_hbm.at[s, 0], kbuf)
        pltpu.sync_copy(a_hbm.at[s, 0], abuf)

        def prep(ci, kk):
            off = ci * _LANES
            new_kk = []
            for k in range(_TOPK):
                va = abuf[k, pl.ds(off, _LANES)]
                vk = kbuf[k, pl.ds(off, _LANES)]
                akbuf[k, pl.ds(off, _LANES)] = va * vk
                a2buf[k, pl.ds(off, _LANES)] = va * va
                new_kk.append(kk[k] + vk * vk)
            return tuple(new_kk)
        kkv = lax.fori_loop(0, _NCH, prep, tuple(zero for _ in range(_TOPK)))
        kk_v = [_lane_sum(kkv[k]) for k in range(_TOPK)]

        def bg_body(bg, _):
            bl = bg * 2

            def c_body(ci, accs):
                off = ci * _LANES
                vx0 = xbuf[bl, pl.ds(off, _LANES)]
                vx1 = xbuf[bl + 1, pl.ds(off, _LANES)]
                w0 = vx0 * vx0
                w1 = vx1 * vx1
                out = []
                for k in range(_TOPK):
                    vak = akbuf[k, pl.ds(off, _LANES)]
                    va2 = a2buf[k, pl.ds(off, _LANES)]
                    n0, n1, d0, d1 = accs[k]
                    out.append((n0 + vx0 * vak, n1 + vx1 * vak,
                                d0 + w0 * va2, d1 + w1 * va2))
                return tuple(out)

            init = tuple((zero, zero, zero, zero) for _ in range(_TOPK))
            accs = lax.fori_loop(0, _NCH, c_body, init)
            lane0 = lax.iota(jnp.int32, _LANES) == 0
            for k in range(_TOPK):
                n0, n1, d0, d1 = accs[k]
                base = k * _BH + bl
                plsc.store_scatter(numbuf, [_splat(base)],
                                   _lane_sum(n0), mask=lane0)
                plsc.store_scatter(numbuf, [_splat(base + 1)],
                                   _lane_sum(n1), mask=lane0)
                plsc.store_scatter(denbuf, [_splat(base)],
                                   _lane_sum(d0), mask=lane0)
                plsc.store_scatter(denbuf, [_splat(base + 1)],
                                   _lane_sum(d1), mask=lane0)
            return 0

        lax.fori_loop(0, _BH // 2, bg_body, 0)

        iota = lax.iota(jnp.int32, _LANES)
        loss_vec = zero
        for k in range(_TOPK):
            vkk = kk_v[k]
            for h in range(_BH // _LANES):
                off = k * _BH + h * _LANES
                vn = numbuf[pl.ds(off, _LANES)]
                vd = denbuf[pl.ds(off, _LANES)]
                t = jnp.maximum(vd * vkk, jnp.float32(1e-24))
                loss_vec = loss_vec + vn * _rsqrt_vec(t)

        def gram_penalty(buf):
            def gbody(ci, accs):
                off = ci * _LANES
                rows = [buf[k, pl.ds(off, _LANES)] for k in range(_TOPK)]
                out = []
                idx = 0
                for i in range(_TOPK):
                    for j in range(i, _TOPK):
                        out.append(accs[idx] + rows[i] * rows[j])
                        idx += 1
                return tuple(out)
            npair = (_TOPK * (_TOPK + 1)) // 2
            accs = lax.fori_loop(0, _NCH, gbody,
                                 tuple(zero for _ in range(npair)))
            sums = [_lane_sum(a) for a in accs]
            diag = {}
            off_pairs = []
            idx = 0
            for i in range(_TOPK):
                for j in range(i, _TOPK):
                    if i == j:
                        diag[i] = sums[idx]
                    else:
                        off_pairs.append((i, j, sums[idx]))
                    idx += 1
            o = zero
            for i, j, sij in off_pairs:
                o = o + jnp.float32(2.0) * (sij * sij) / (diag[i] * diag[j])
            return o / jnp.float32(_TOPK * _TOPK) * jnp.float32(1e-06)

        o_val = gram_penalty(kbuf) + gram_penalty(abuf)
        orth_gated = jnp.where(_splat(c) == 0, o_val, zero)

        orth_vec = jnp.where(iota == 0, orth_gated, zero)
        partbuf[pl.ds(0, _LANES)] = loss_vec
        partbuf[pl.ds(_LANES, _LANES)] = orth_vec
        pltpu.sync_copy(partbuf, shared.at[s])

    plsc.subcore_barrier()

    @pl.when(s == 0)
    def _reduce():
        pltpu.sync_copy(shared, gathbuf)
        lv = zero
        ov = zero
        for sl in range(_L):
            lv = lv + gathbuf[sl, pl.ds(0, _LANES)]
            ov = ov + gathbuf[sl, pl.ds(_LANES, _LANES)]
        cos_sum = _lane_sum(lv)
        orth_sum = _lane_sum(ov)
        p_part = jnp.float32(_LOSS_W) * (
            jnp.float32(_L * _BH * _TOPK) - cos_sum)
        o_part = jnp.float32(_ORTH_MU) * orth_sum
        iota = lax.iota(jnp.int32, _LANES)
        res = jnp.where(iota == 0, p_part,
                        jnp.where(iota == 1, o_part, zero))
        outbuf[...] = res
        pltpu.sync_copy(outbuf, out_hbm.at[c])


def _sc_losses(x_query, ek_full, ea_full):
    mesh = plsc.VectorSubcoreMesh(core_axis_name="c", subcore_axis_name="s")
    f32 = jnp.float32
    kern = pl.kernel(
        _sc_loss_body,
        mesh=mesh,
        compiler_params=pltpu.CompilerParams(needs_layout_passes=False),
        out_type=jax.ShapeDtypeStruct((_NC, _LANES), f32),
        scratch_types=[
            pltpu.VMEM((_BH, _D), f32),
            pltpu.VMEM((_TOPK, _D), f32),
            pltpu.VMEM((_TOPK, _D), f32),
            pltpu.VMEM((_TOPK, _D), f32),
            pltpu.VMEM((_TOPK, _D), f32),
            pltpu.VMEM((_TOPK * _BH,), f32),
            pltpu.VMEM((_TOPK * _BH,), f32),
            pltpu.VMEM((128,), f32),
            pltpu.VMEM((16, 128), f32),
            pltpu.VMEM((_LANES,), f32),
            pltpu.VMEM_SHARED((16, 128), f32),
        ],
    )
    return kern(x_query, ek_full, ea_full)


def _bcast_body(p_ref, out_ref):
    pb = p_ref[0, 0].reshape(1, p_ref.shape[2] * p_ref.shape[3],
                             p_ref.shape[4])
    out_ref[...] = jnp.broadcast_to(pb, out_ref.shape)


def kernel(x_query, vis_mark, train, e_p, e_k, e_a):
    L, T, topk, plen, D = e_p.shape
    Bq = x_query.shape[0]

    lv = _sc_losses(x_query, e_k, e_a)

    out = pl.pallas_call(
        _bcast_body,
        grid=(L,),
        in_specs=[pl.BlockSpec((1, 1, topk, plen, D),
                               lambda l: (l, 0, 0, 0, 0))],
        out_specs=pl.BlockSpec((1, Bq, topk * plen, D),
                               lambda l: (l, 0, 0, 0)),
        out_shape=jax.ShapeDtypeStruct((L, Bq, topk * plen, D), jnp.float32),
    )(e_p)
    p_loss = lv[0, 0] + lv[1, 0]
    orth_loss = lv[0, 1] + lv[1, 1]
    return out, p_loss, orth_loss


# speedup vs baseline: 1.0596x; 1.0596x over previous
import jax
import jax.numpy as jnp
from jax import lax
from jax.experimental import pallas as pl
from jax.experimental.pallas import tpu as pltpu
from jax.experimental.pallas import tpu_sc as plsc

_LOSS_W = 0.5
_ORTH_MU = 0.1
_NC = 2
_LANES = 16

_L = 12
_B = 64
_D = 768
_TOPK = 5
_NCH = _D // _LANES
_BH = _B // 2


def _splat(s):
    return lax.broadcast_in_dim(s, (_LANES,), ())


def _rsqrt_vec(v):
    i = plsc.bitcast(v, jnp.int32)
    i = jnp.int32(0x5F3759DF) - lax.shift_right_logical(
        i, jnp.full((_LANES,), 1, jnp.int32))
    y = plsc.bitcast(i, jnp.float32)
    for _ in range(3):
        y = y * (jnp.float32(1.5) - jnp.float32(0.5) * v * y * y)
    return y


def _lane_sum(v):
    return _splat(jnp.sum(v))


def _sc_loss_body(x_hbm, k_hbm, a_hbm, out_hbm,
                  xbuf, kbuf, abuf, akbuf, a2buf, numbuf, denbuf,
                  partbuf, gathbuf, outbuf, shared):
    c = lax.axis_index("c")
    s = lax.axis_index("s")
    zero = jnp.zeros((_LANES,), jnp.float32)

    @pl.when(s < _L)
    def _compute():
        b0 = c * _BH
        pltpu.sync_copy(x_hbm.at[pl.ds(b0, _BH), s], xbuf)
        pltpu.sync_copy(k_hbm.at[s], kbuf)
        pltpu.sync_copy(a_hbm.at[s], abuf)

        def prep(ci, kk):
            off = ci * _LANES
            new_kk = []
            for k in range(_TOPK):
                va = abuf[k, pl.ds(off, _LANES)]
                vk = kbuf[k, pl.ds(off, _LANES)]
                akbuf[k, pl.ds(off, _LANES)] = va * vk
                a2buf[k, pl.ds(off, _LANES)] = va * va
                new_kk.append(kk[k] + vk * vk)
            return tuple(new_kk)
        kkv = lax.fori_loop(0, _NCH, prep, tuple(zero for _ in range(_TOPK)))
        kk_v = [_lane_sum(kkv[k]) for k in range(_TOPK)]

        def bg_body(bg, _):
            bl = bg * 2

            def c_body(ci, accs):
                off = ci * _LANES
                vx0 = xbuf[bl, pl.ds(off, _LANES)]
                vx1 = xbuf[bl + 1, pl.ds(off, _LANES)]
                w0 = vx0 * vx0
                w1 = vx1 * vx1
                out = []
                for k in range(_TOPK):
                    vak = akbuf[k, pl.ds(off, _LANES)]
                    va2 = a2buf[k, pl.ds(off, _LANES)]
                    n0, n1, d0, d1 = accs[k]
                    out.append((n0 + vx0 * vak, n1 + vx1 * vak,
                                d0 + w0 * va2, d1 + w1 * va2))
                return tuple(out)

            init = tuple((zero, zero, zero, zero) for _ in range(_TOPK))
            accs = lax.fori_loop(0, _NCH, c_body, init)
            lane0 = lax.iota(jnp.int32, _LANES) == 0
            for k in range(_TOPK):
                n0, n1, d0, d1 = accs[k]
                base = k * _BH + bl
                plsc.store_scatter(numbuf, [_splat(base)],
                                   _lane_sum(n0), mask=lane0)
                plsc.store_scatter(numbuf, [_splat(base + 1)],
                                   _lane_sum(n1), mask=lane0)
                plsc.store_scatter(denbuf, [_splat(base)],
                                   _lane_sum(d0), mask=lane0)
                plsc.store_scatter(denbuf, [_splat(base + 1)],
                                   _lane_sum(d1), mask=lane0)
            return 0

        lax.fori_loop(0, _BH // 2, bg_body, 0)

        iota = lax.iota(jnp.int32, _LANES)
        loss_vec = zero
        for k in range(_TOPK):
            vkk = kk_v[k]
            for h in range(_BH // _LANES):
                off = k * _BH + h * _LANES
                vn = numbuf[pl.ds(off, _LANES)]
                vd = denbuf[pl.ds(off, _LANES)]
                t = jnp.maximum(vd * vkk, jnp.float32(1e-24))
                loss_vec = loss_vec + vn * _rsqrt_vec(t)

        def gram_penalty(buf):
            def gbody(ci, accs):
                off = ci * _LANES
                rows = [buf[k, pl.ds(off, _LANES)] for k in range(_TOPK)]
                out = []
                idx = 0
                for i in range(_TOPK):
                    for j in range(i, _TOPK):
                        out.append(accs[idx] + rows[i] * rows[j])
                        idx += 1
                return tuple(out)
            npair = (_TOPK * (_TOPK + 1)) // 2
            accs = lax.fori_loop(0, _NCH, gbody,
                                 tuple(zero for _ in range(npair)))
            sums = [_lane_sum(a) for a in accs]
            diag = {}
            off_pairs = []
            idx = 0
            for i in range(_TOPK):
                for j in range(i, _TOPK):
                    if i == j:
                        diag[i] = sums[idx]
                    else:
                        off_pairs.append((i, j, sums[idx]))
                    idx += 1
            o = zero
            for i, j, sij in off_pairs:
                o = o + jnp.float32(2.0) * (sij * sij) / (diag[i] * diag[j])
            return o / jnp.float32(_TOPK * _TOPK) * jnp.float32(1e-06)

        o_val = gram_penalty(kbuf) + gram_penalty(abuf)
        orth_gated = jnp.where(_splat(c) == 0, o_val, zero)

        orth_vec = jnp.where(iota == 0, orth_gated, zero)
        partbuf[pl.ds(0, _LANES)] = loss_vec
        partbuf[pl.ds(_LANES, _LANES)] = orth_vec
        pltpu.sync_copy(partbuf, shared.at[s])

    plsc.subcore_barrier()

    @pl.when(s == 0)
    def _reduce():
        pltpu.sync_copy(shared, gathbuf)
        lv = zero
        ov = zero
        for sl in range(_L):
            lv = lv + gathbuf[sl, pl.ds(0, _LANES)]
            ov = ov + gathbuf[sl, pl.ds(_LANES, _LANES)]
        cos_sum = _lane_sum(lv)
        orth_sum = _lane_sum(ov)
        p_part = jnp.float32(_LOSS_W) * (
            jnp.float32(_L * _BH * _TOPK) - cos_sum)
        o_part = jnp.float32(_ORTH_MU) * orth_sum
        iota = lax.iota(jnp.int32, _LANES)
        res = jnp.where(iota == 0, p_part,
                        jnp.where(iota == 1, o_part, zero))
        outbuf[...] = res
        pltpu.sync_copy(outbuf, out_hbm.at[c])


def _sc_losses(x_query, k0, a0):
    mesh = plsc.VectorSubcoreMesh(core_axis_name="c", subcore_axis_name="s")
    f32 = jnp.float32
    kern = pl.kernel(
        _sc_loss_body,
        mesh=mesh,
        compiler_params=pltpu.CompilerParams(needs_layout_passes=False),
        out_type=jax.ShapeDtypeStruct((_NC, _LANES), f32),
        scratch_types=[
            pltpu.VMEM((_BH, _D), f32),
            pltpu.VMEM((_TOPK, _D), f32),
            pltpu.VMEM((_TOPK, _D), f32),
            pltpu.VMEM((_TOPK, _D), f32),
            pltpu.VMEM((_TOPK, _D), f32),
            pltpu.VMEM((_TOPK * _BH,), f32),
            pltpu.VMEM((_TOPK * _BH,), f32),
            pltpu.VMEM((128,), f32),
            pltpu.VMEM((16, 128), f32),
            pltpu.VMEM((_LANES,), f32),
            pltpu.VMEM_SHARED((16, 128), f32),
        ],
    )
    return kern(x_query, k0, a0)


def _bcast_body(p_ref, out_ref):
    pb = p_ref[0, 0].reshape(1, p_ref.shape[2] * p_ref.shape[3],
                             p_ref.shape[4])
    out_ref[...] = jnp.broadcast_to(pb, out_ref.shape)


def kernel(x_query, vis_mark, train, e_p, e_k, e_a):
    L, T, topk, plen, D = e_p.shape
    Bq = x_query.shape[0]

    lv = _sc_losses(x_query, e_k[:, 0], e_a[:, 0])

    out = pl.pallas_call(
        _bcast_body,
        grid=(L,),
        in_specs=[pl.BlockSpec((1, 1, topk, plen, D),
                               lambda l: (l, 0, 0, 0, 0))],
        out_specs=pl.BlockSpec((1, Bq, topk * plen, D),
                               lambda l: (l, 0, 0, 0)),
        out_shape=jax.ShapeDtypeStruct((L, Bq, topk * plen, D), jnp.float32),
    )(e_p)
    p_loss = lv[0, 0] + lv[1, 0]
    orth_loss = lv[0, 1] + lv[1, 1]
    return out, p_loss, orth_loss
